# baseline (speedup 1.0000x reference)
import functools

import jax
import jax.numpy as jnp
from jax import lax
from jax.experimental import pallas as pl
from jax.experimental.pallas import tpu as pltpu
from jax.experimental.pallas import tpu_sc as plsc

N_ROWS = 1_000_000
D = 32
B = 16_384

NUM_CORES = 2
NUM_TILES = 16
LANES = 16

ROWS_PER_CORE = N_ROWS // NUM_CORES
CHUNK = 20_000
CHUNKS_PER_CORE = ROWS_PER_CORE // CHUNK
ROWS_PER_TILE = 1248
TAIL_ROWS = CHUNK - NUM_TILES * ROWS_PER_TILE
IDX_PER_TILE = B // NUM_TILES
IDX_GROUPS = IDX_PER_TILE // 128
VECS_PER_TILE = IDX_PER_TILE // LANES


def _body(inp_hbm, idx_hbm, src_hbm, out_hbm, idx_v, src_v, ridx_v, buf):
    c = lax.axis_index("c")
    s = lax.axis_index("s")

    pltpu.sync_copy(idx_hbm.at[pl.ds(s * IDX_GROUPS, IDX_GROUPS)], idx_v)
    pltpu.sync_copy(src_hbm.at[pl.ds(s * IDX_PER_TILE, IDX_PER_TILE)], src_v)

    def chunk_body(ci, carry):
        base = c * ROWS_PER_CORE + ci * CHUNK

        pltpu.sync_copy(
            inp_hbm.at[pl.ds(pl.multiple_of(base + s * ROWS_PER_TILE, 8),
                             ROWS_PER_TILE)],
            buf.at[pl.ds(pl.multiple_of(s * ROWS_PER_TILE, 8), ROWS_PER_TILE)],
        )

        @pl.when(s == 0)
        def _copy_in_tail():
            pltpu.sync_copy(
                inp_hbm.at[pl.ds(pl.multiple_of(base + NUM_TILES * ROWS_PER_TILE, 8),
                                 TAIL_ROWS)],
                buf.at[pl.ds(NUM_TILES * ROWS_PER_TILE, TAIL_ROWS)],
            )

        plsc.subcore_barrier()

        dummy = CHUNK + s
        for v in range(VECS_PER_TILE):
            r, col = v // (128 // LANES), (v % (128 // LANES)) * LANES
            iv = idx_v[r, pl.ds(col, LANES)]
            local = iv - base
            ok = (local >= 0) & (local < CHUNK)
            ridx_v[r, pl.ds(col, LANES)] = jnp.where(ok, local, dummy)

        for j in range(IDX_GROUPS):
            pltpu.sync_copy(
                src_v.at[pl.ds(j * 128, 128)],
                buf.at[ridx_v.at[j]],
                add=True,
            )
        plsc.subcore_barrier()

        pltpu.sync_copy(
            buf.at[pl.ds(pl.multiple_of(s * ROWS_PER_TILE, 8), ROWS_PER_TILE)],
            out_hbm.at[pl.ds(pl.multiple_of(base + s * ROWS_PER_TILE, 8),
                             ROWS_PER_TILE)],
        )

        @pl.when(s == 0)
        def _copy_out_tail():
            pltpu.sync_copy(
                buf.at[pl.ds(NUM_TILES * ROWS_PER_TILE, TAIL_ROWS)],
                out_hbm.at[pl.ds(pl.multiple_of(base + NUM_TILES * ROWS_PER_TILE, 8),
                                 TAIL_ROWS)],
            )

        plsc.subcore_barrier()
        return carry

    lax.fori_loop(0, CHUNKS_PER_CORE, chunk_body, 0)


_index_add_sc = functools.partial(
    pl.kernel,
    out_type=jax.ShapeDtypeStruct((N_ROWS, D), jnp.float32),
    mesh=plsc.VectorSubcoreMesh(core_axis_name="c", subcore_axis_name="s"),
    scratch_types=[
        pltpu.VMEM((IDX_GROUPS, 128), jnp.int32),
        pltpu.VMEM((IDX_PER_TILE, D), jnp.float32),
        pltpu.VMEM((IDX_GROUPS, 128), jnp.int32),
        pltpu.VMEM_SHARED((CHUNK + NUM_TILES, D), jnp.float32),
    ],
    compiler_params=pltpu.CompilerParams(use_tc_tiling_on_sc=False),
)(_body)


def kernel(input_tensor, index, source_tensor):
    idx = index.astype(jnp.int32).reshape(B // 128, 128)
    return _index_add_sc(input_tensor, idx, source_tensor)
